# baseline (speedup 1.0000x reference)
import functools

import jax
import jax.numpy as jnp
from jax import lax
from jax.experimental import pallas as pl
from jax.experimental.pallas import tpu as pltpu
from jax.experimental.pallas import tpu_sc as plsc

K = 8192
D = 256
NB = 8
T = 24 * 24
N = NB * T
KB = 1024
NKB = K // KB
NW = 32
BPW = N // NW
CH = 72
NCH = BPW // CH
COMMIT = 0.25


def _argmin_body(x_ref, e_ref, idx_ref, loss_ref, rv, ri):
    kb = pl.program_id(0)
    E = e_ref[...]
    e2 = jnp.sum(E * E, axis=1, keepdims=True)
    Xm2 = x_ref[...] * (-2.0)
    S = lax.dot_general(E, Xm2, (((1,), (0,)), ((), ())),
                        preferred_element_type=jnp.float32)
    d = e2 + S
    bm = jnp.min(d, axis=0, keepdims=True)
    iif = lax.broadcasted_iota(jnp.int32, (KB, N), 0).astype(jnp.float32)
    bi = jnp.min(jnp.where(d == bm, iif, float(K)), axis=0,
                 keepdims=True)

    @pl.when(kb == 0)
    def _():
        rv[...] = bm
        ri[...] = bi.astype(jnp.int32)

    @pl.when(kb > 0)
    def _():
        take = bm < rv[...]
        ri[...] = jnp.where(take, bi.astype(jnp.int32) + kb * KB, ri[...])
        rv[...] = jnp.where(take, bm, rv[...])

    @pl.when(kb == NKB - 1)
    def _():
        x2 = jnp.sum(x_ref[...] * x_ref[...], axis=0, keepdims=True)
        idx_ref[...] = ri[...]
        loss_ref[0, 0] = jnp.sum(rv[...] + x2) * ((1.0 + COMMIT) / (N * D))


def _tc_argmin(xt, emb):
    return pl.pallas_call(
        _argmin_body,
        grid=(NKB,),
        in_specs=[
            pl.BlockSpec((D, N), lambda kb: (0, 0)),
            pl.BlockSpec((KB, D), lambda kb: (kb, 0)),
        ],
        out_specs=[
            pl.BlockSpec((1, N), lambda kb: (0, 0)),
            pl.BlockSpec(memory_space=pltpu.SMEM),
        ],
        out_shape=[
            jax.ShapeDtypeStruct((1, N), jnp.int32),
            jax.ShapeDtypeStruct((1, 1), jnp.float32),
        ],
        scratch_shapes=[
            pltpu.VMEM((1, N), jnp.float32),
            pltpu.VMEM((1, N), jnp.int32),
        ],
    )(xt, emb)


def _sc_body(emb_hbm, idx_hbm, z_hbm, q_hbm, cnt_hbm,
             idx_v, rows_v, ones_v, hist_sh, sem):
    cid = lax.axis_index("c")
    sid = lax.axis_index("s")
    wid = sid * 2 + cid
    base = wid * BPW
    for j in range(NCH):
        pltpu.sync_copy(idx_hbm.at[pl.ds(base + j * CH, CH)], idx_v.at[j])
    cps = [pltpu.async_copy(emb_hbm.at[idx_v.at[j]],
                            rows_v.at[pl.ds(j * CH, CH)], sem)
           for j in range(NCH)]
    for cp in cps:
        cp.wait()
    pltpu.sync_copy(rows_v, q_hbm.at[pl.ds(base, BPW)])
    for j in range(BPW // 16):
        ones_v[pl.ds(j * 16, 16)] = jnp.ones((16,), jnp.float32)

    @pl.when(sid == 0)
    def _():
        pltpu.sync_copy(z_hbm, hist_sh)

    plsc.subcore_barrier()
    for j in range(NCH):
        pltpu.sync_copy(ones_v.at[pl.ds(0, CH)],
                        hist_sh.at[idx_v.at[j]], add=True)
    plsc.subcore_barrier()

    @pl.when(sid == 0)
    def _():
        pltpu.sync_copy(hist_sh, cnt_hbm.at[cid])


def _sc_gather_hist(emb, idx_flat, zeros):
    mesh = plsc.VectorSubcoreMesh(core_axis_name="c", subcore_axis_name="s")
    run = functools.partial(
        pl.kernel,
        out_type=[
            jax.ShapeDtypeStruct((N, D), jnp.float32),
            jax.ShapeDtypeStruct((2, K), jnp.float32),
        ],
        mesh=mesh,
        scratch_types=[
            pltpu.VMEM((NCH, CH), jnp.int32),
            pltpu.VMEM((BPW, D), jnp.float32),
            pltpu.VMEM((BPW,), jnp.float32),
            pltpu.VMEM_SHARED((K,), jnp.float32),
            pltpu.SemaphoreType.DMA,
        ],
    )(_sc_body)
    return run(emb, idx_flat, zeros)


def _perp_body(cnt_ref, out_ref):
    c = cnt_ref[0:1, :] + cnt_ref[1:2, :]
    p = c * (1.0 / N)
    ent = jnp.sum(p * jnp.log(p + 1e-10))
    out_ref[0, 0] = jnp.exp(-ent)


def _tc_perp(cnt):
    return pl.pallas_call(
        _perp_body,
        in_specs=[pl.BlockSpec((2, K), lambda: (0, 0))],
        out_specs=pl.BlockSpec(memory_space=pltpu.SMEM),
        out_shape=jax.ShapeDtypeStruct((1, 1), jnp.float32),
        grid=(),
    )(cnt)


def kernel(inputs, embeddings):
    B_, C, H, W = inputs.shape
    xt = jnp.transpose(inputs.reshape(NB, D, T), (1, 0, 2)).reshape(D, N)
    idx2d, loss_s = _tc_argmin(xt, embeddings)
    idx_flat = idx2d.reshape(N)
    zeros = jnp.zeros((K,), jnp.float32)
    q, cnt = _sc_gather_hist(embeddings, idx_flat, zeros)
    perp_s = _tc_perp(cnt)
    quantized = jnp.transpose(q.reshape(NB, H, W, D), (0, 3, 1, 2))
    return quantized, loss_s[0, 0], perp_s[0, 0], idx2d.reshape(NB, H, W)


# speedup vs baseline: 1.0960x; 1.0960x over previous
import functools

import jax
import jax.numpy as jnp
from jax import lax
from jax.experimental import pallas as pl
from jax.experimental.pallas import tpu as pltpu
from jax.experimental.pallas import tpu_sc as plsc

K = 8192
D = 256
NB = 8
T = 24 * 24
N = NB * T
KB = 512
NKB = K // KB
NW = 32
BPW = N // NW
CH = 72
NCH = BPW // CH
COMMIT = 0.25


def _argmin_body(x_ref, e_ref, idx_ref, loss_ref, rv, ri, ti):
    kb = pl.program_id(0)
    E = e_ref[...]
    e2 = jnp.sum(E * E, axis=1, keepdims=True)
    Xm2 = x_ref[...] * (-2.0)
    S = lax.dot_general(E, Xm2, (((1,), (0,)), ((), ())),
                        preferred_element_type=jnp.float32)
    d = e2 + S
    bm = jnp.min(d, axis=0, keepdims=True)
    m_f = jnp.where(d == bm, 1.0, 0.0)
    lane = lax.broadcasted_iota(jnp.int32, (8, KB), 1)
    sub = lax.broadcasted_iota(jnp.int32, (8, KB), 0)
    lo = (lane & 63).astype(jnp.float32)
    hi = (lane >> 6).astype(jnp.float32)
    one = jnp.float32(1.0)
    lhs = jnp.where(sub == 0, lo,
                    jnp.where(sub == 1, hi,
                              jnp.where(sub == 2, one, 0.0)))
    sc = lax.dot_general(lhs, m_f, (((1,), (0,)), ((), ())),
                         preferred_element_type=jnp.float32)
    bi = jnp.floor(sc[0:1] + 64.0 * sc[1:2] + 0.5)
    cnt = sc[2:3]

    @pl.when(jnp.max(cnt) > 1.5)
    def _():
        iif = lax.broadcasted_iota(jnp.int32, (KB, N), 0).astype(jnp.float32)
        ti[...] = jnp.min(jnp.where(d == bm, iif, float(K)), axis=0,
                          keepdims=True)

    @pl.when(jnp.max(cnt) <= 1.5)
    def _():
        ti[...] = bi

    bi = ti[...]

    @pl.when(kb == 0)
    def _():
        rv[...] = bm
        ri[...] = bi.astype(jnp.int32)

    @pl.when(kb > 0)
    def _():
        take = bm < rv[...]
        ri[...] = jnp.where(take, bi.astype(jnp.int32) + kb * KB, ri[...])
        rv[...] = jnp.where(take, bm, rv[...])

    @pl.when(kb == NKB - 1)
    def _():
        x2 = jnp.sum(x_ref[...] * x_ref[...], axis=0, keepdims=True)
        idx_ref[...] = ri[...]
        loss_ref[0, 0] = jnp.sum(rv[...] + x2) * ((1.0 + COMMIT) / (N * D))


def _tc_argmin(xt, emb):
    return pl.pallas_call(
        _argmin_body,
        grid=(NKB,),
        in_specs=[
            pl.BlockSpec((D, N), lambda kb: (0, 0)),
            pl.BlockSpec((KB, D), lambda kb: (kb, 0)),
        ],
        out_specs=[
            pl.BlockSpec((1, N), lambda kb: (0, 0)),
            pl.BlockSpec(memory_space=pltpu.SMEM),
        ],
        out_shape=[
            jax.ShapeDtypeStruct((1, N), jnp.int32),
            jax.ShapeDtypeStruct((1, 1), jnp.float32),
        ],
        scratch_shapes=[
            pltpu.VMEM((1, N), jnp.float32),
            pltpu.VMEM((1, N), jnp.int32),
            pltpu.VMEM((1, N), jnp.float32),
        ],
    )(xt, emb)


def _sc_body(emb_hbm, idx_hbm, z_hbm, q_hbm, cnt_hbm,
             idx_v, rows_v, ones_v, hist_sh, sem):
    cid = lax.axis_index("c")
    sid = lax.axis_index("s")
    wid = sid * 2 + cid
    base = wid * BPW
    for j in range(NCH):
        pltpu.sync_copy(idx_hbm.at[pl.ds(base + j * CH, CH)], idx_v.at[j])
    cps = [pltpu.async_copy(emb_hbm.at[idx_v.at[j]],
                            rows_v.at[pl.ds(j * CH, CH)], sem)
           for j in range(NCH)]
    for cp in cps:
        cp.wait()
    pltpu.sync_copy(rows_v, q_hbm.at[pl.ds(base, BPW)])
    for j in range(BPW // 16):
        ones_v[pl.ds(j * 16, 16)] = jnp.ones((16,), jnp.float32)

    @pl.when(sid == 0)
    def _():
        pltpu.sync_copy(z_hbm, hist_sh)

    plsc.subcore_barrier()
    for j in range(NCH):
        pltpu.sync_copy(ones_v.at[pl.ds(0, CH)],
                        hist_sh.at[idx_v.at[j]], add=True)
    plsc.subcore_barrier()

    @pl.when(sid == 0)
    def _():
        pltpu.sync_copy(hist_sh, cnt_hbm.at[cid])


def _sc_gather_hist(emb, idx_flat, zeros):
    mesh = plsc.VectorSubcoreMesh(core_axis_name="c", subcore_axis_name="s")
    run = functools.partial(
        pl.kernel,
        out_type=[
            jax.ShapeDtypeStruct((N, D), jnp.float32),
            jax.ShapeDtypeStruct((2, K), jnp.float32),
        ],
        mesh=mesh,
        scratch_types=[
            pltpu.VMEM((NCH, CH), jnp.int32),
            pltpu.VMEM((BPW, D), jnp.float32),
            pltpu.VMEM((BPW,), jnp.float32),
            pltpu.VMEM_SHARED((K,), jnp.float32),
            pltpu.SemaphoreType.DMA,
        ],
    )(_sc_body)
    return run(emb, idx_flat, zeros)


def _perp_body(cnt_ref, out_ref):
    c = cnt_ref[0:1, :] + cnt_ref[1:2, :]
    p = c * (1.0 / N)
    ent = jnp.sum(p * jnp.log(p + 1e-10))
    out_ref[0, 0] = jnp.exp(-ent)


def _tc_perp(cnt):
    return pl.pallas_call(
        _perp_body,
        in_specs=[pl.BlockSpec((2, K), lambda: (0, 0))],
        out_specs=pl.BlockSpec(memory_space=pltpu.SMEM),
        out_shape=jax.ShapeDtypeStruct((1, 1), jnp.float32),
        grid=(),
    )(cnt)


def kernel(inputs, embeddings):
    B_, C, H, W = inputs.shape
    xt = jnp.transpose(inputs.reshape(NB, D, T), (1, 0, 2)).reshape(D, N)
    idx2d, loss_s = _tc_argmin(xt, embeddings)
    idx_flat = idx2d.reshape(N)
    zeros = jnp.zeros((K,), jnp.float32)
    q, cnt = _sc_gather_hist(embeddings, idx_flat, zeros)
    perp_s = _tc_perp(cnt)
    quantized = jnp.transpose(q.reshape(NB, H, W, D), (0, 3, 1, 2))
    return quantized, loss_s[0, 0], perp_s[0, 0], idx2d.reshape(NB, H, W)
